# baseline (speedup 1.0000x reference)
import jax
import jax.numpy as jnp
from jax import lax
from jax.experimental import pallas as pl
from jax.experimental.pallas import tpu as pltpu
from jax.experimental.pallas import tpu_sc as plsc

B, T, D, E = 2, 2048, 1024, 8
N = B * T
NC, NS, L = 2, 16, 16
NW = NC * NS
TPW = N // NW
DE = D * E


def _sc_body(experts_hbm, gate_hbm, out_hbm, gate_v, eidx_v, slab_v, row_v,
             gsem, isem, osem):
    wid = lax.axis_index("s") * NC + lax.axis_index("c")
    base = wid * TPW
    pltpu.async_copy(gate_hbm.at[pl.ds(base * E, TPW * E)], gate_v, gsem).wait()

    lane = lax.iota(jnp.int32, L)

    @pl.loop(0, TPW // L)
    def _(c):
        tok0 = lane * E + c * (L * E)
        bv = plsc.load_gather(gate_v, [tok0])
        bi = jnp.zeros((L,), jnp.int32)
        for e in range(1, E):
            v = plsc.load_gather(gate_v, [tok0 + e])
            better = v > bv
            bv = jnp.where(better, v, bv)
            bi = jnp.where(better, jnp.full((L,), e, jnp.int32), bi)
        eidx_v[pl.ds(c * L, L)] = bi

    pltpu.async_copy(experts_hbm.at[base], slab_v.at[0], isem)
    pltpu.async_copy(experts_hbm.at[base + 1], slab_v.at[1], isem)

    @pl.loop(0, TPW, step=2)
    def _(t0):
        for b in range(2):
            t = t0 + b
            pltpu.make_async_copy(experts_hbm.at[base], slab_v.at[b],
                                  isem).wait()
            @pl.when(t0 >= 2)
            def _():
                pltpu.make_async_copy(out_hbm.at[pl.ds(0, D)], row_v.at[b],
                                      osem).wait()

            e_splat = plsc.load_gather(eidx_v, [jnp.full((L,), t, jnp.int32)])
            slab = slab_v.at[b]
            row = row_v.at[b]

            @pl.loop(0, D // L)
            def _(c):
                idx = (lane + c * L) * E + e_splat
                row[pl.ds(c * L, L)] = plsc.load_gather(slab, [idx])

            @pl.when(t0 + b + 2 < TPW)
            def _():
                pltpu.async_copy(experts_hbm.at[base + t + 2], slab_v.at[b],
                                 isem)

            pltpu.async_copy(row_v.at[b], out_hbm.at[pl.ds((base + t) * D, D)],
                             osem)

    for b in range(2):
        pltpu.make_async_copy(out_hbm.at[pl.ds(0, D)], row_v.at[b],
                              osem).wait()


@jax.jit
def _run(experts, gate):
    ef = experts.reshape(N, DE)
    gf = gate.reshape(N * E)
    mesh = plsc.VectorSubcoreMesh(core_axis_name="c", subcore_axis_name="s")
    out = pl.kernel(
        _sc_body,
        out_type=jax.ShapeDtypeStruct((N * D,), jnp.float32),
        mesh=mesh,
        compiler_params=pltpu.CompilerParams(use_tc_tiling_on_sc=False,
                                             needs_layout_passes=False),
        scratch_types=[
            pltpu.VMEM((TPW * E,), jnp.float32),
            pltpu.VMEM((TPW,), jnp.int32),
            pltpu.VMEM((2, DE), jnp.float32),
            pltpu.VMEM((2, D), jnp.float32),
            pltpu.SemaphoreType.DMA,
            pltpu.SemaphoreType.DMA,
            pltpu.SemaphoreType.DMA,
        ],
    )(ef, gf)
    return out.reshape(B, T, D)


def kernel(experts, gate):
    return _run(experts, gate)
